# baseline (speedup 1.0000x reference)
import functools

import jax
import jax.numpy as jnp
from jax import lax
from jax.experimental import pallas as pl
from jax.experimental.pallas import tpu as pltpu
from jax.experimental.pallas import tpu_sc as plsc

B, S, F = 4, 512, 8
V, DM = 32000, 768
FV = 1000
E, DSW, DFF, L = 8, 256, 1024, 2
EPS = 1e-5

NC, NS = 2, 16
NW = NC * NS
NTOK = B * S
HALVES = 2
BH = B // HALVES
NTOKH = NTOK // HALVES
TPW = NTOKH // NW
SUB = 4
NSUB = TPW // SUB
LANES = 16
NBUF = 3


def _sc_embed_body(tok_hbm, feat_hbm, ttab_hbm, ftab_hbm,
                   tok_out, fsum_out,
                   tok_idx_v, feat_idx_v, acc_v, s0_v, s1_v, s2_v,
                   gsem0, gsem1, gsem2, tsem):
    wid = lax.axis_index("s") * NC + lax.axis_index("c")
    base = wid * TPW

    icopy = pltpu.async_copy(feat_hbm.at[pl.ds(wid * NSUB, NSUB)],
                             feat_idx_v, gsem0)
    pltpu.async_copy(tok_hbm.at[pl.ds(base, TPW)], tok_idx_v, tsem).wait()

    tok_dma = pltpu.async_copy(ttab_hbm.at[tok_idx_v], acc_v, tsem)
    icopy.wait()

    for n in range(NSUB):
        for c in range(SUB * F // LANES):
            v = feat_idx_v[n, pl.ds(c * LANES, LANES)]
            feat_idx_v[n, pl.ds(c * LANES, LANES)] = jnp.where(v == 0, FV, v)

    stages = (s0_v, s1_v, s2_v)
    gsems = (gsem0, gsem1, gsem2)

    dmas = [pltpu.async_copy(ftab_hbm.at[feat_idx_v.at[n]],
                             stages[n], gsems[n]) for n in range(NBUF - 1)]
    tok_dma.wait()
    pltpu.sync_copy(acc_v, tok_out.at[pl.ds(base, TPW)])

    for n in range(NSUB):
        if n + NBUF - 1 < NSUB:
            dmas.append(pltpu.async_copy(
                ftab_hbm.at[feat_idx_v.at[n + NBUF - 1]],
                stages[(n + NBUF - 1) % NBUF], gsems[(n + NBUF - 1) % NBUF]))
        dmas[n].wait()
        stage = stages[n % NBUF]

        def cbody(c, stage=stage, n=n):
            off = c * LANES
            for t in range(SUB):
                s = stage[t * F + 0, pl.ds(off, LANES)]
                for f in range(1, F):
                    s = s + stage[t * F + f, pl.ds(off, LANES)]
                acc_v[n * SUB + t, pl.ds(off, LANES)] = s

        plsc.parallel_loop(0, DM // LANES, 1, unroll=4)(cbody)

    pltpu.sync_copy(acc_v, fsum_out.at[pl.ds(base, TPW)])


def _sc_embed(tokens_flat, feat_idx2d, token_table, ftab_aug):
    mesh = plsc.VectorSubcoreMesh(core_axis_name="c", subcore_axis_name="s",
                                  num_cores=NC, num_subcores=NS)
    k = pl.kernel(
        _sc_embed_body,
        out_type=[jax.ShapeDtypeStruct((NTOKH, DM), jnp.float32),
                  jax.ShapeDtypeStruct((NTOKH, DM), jnp.float32)],
        mesh=mesh,
        scratch_types=[
            pltpu.VMEM((TPW,), jnp.int32),
            pltpu.VMEM((NSUB, SUB * F), jnp.int32),
            pltpu.VMEM((TPW, DM), jnp.float32),
            pltpu.VMEM((SUB * F, DM), jnp.float32),
            pltpu.VMEM((SUB * F, DM), jnp.float32),
            pltpu.VMEM((SUB * F, DM), jnp.float32),
            pltpu.SemaphoreType.DMA,
            pltpu.SemaphoreType.DMA,
            pltpu.SemaphoreType.DMA,
            pltpu.SemaphoreType.DMA,
        ],
    )
    return k(tokens_flat, feat_idx2d, token_table, ftab_aug)


def _lnk(x, g, b):
    mu = jnp.mean(x, axis=-1, keepdims=True)
    msq = jnp.mean(x * x, axis=-1, keepdims=True)
    var = msq - mu * mu
    return (x - mu) * lax.rsqrt(var + EPS) * g + b


def _dot(a, b):
    return lax.dot_general(a, b, (((1,), (0,)), ((), ())),
                           preferred_element_type=jnp.float32)


def _dot_t(a, b):
    return lax.dot_general(a, b, (((0,), (0,)), ((), ())),
                           preferred_element_type=jnp.float32)


def _tc_body(mask_ref, tok_ref, fsum_ref, fg_ref, fb_ref,
             Win_ref, bin_ref, sg_ref, sb_ref,
             Wef_ref, bef_ref, flg_ref, flb_ref,
             Web_ref, beb_ref, blg_ref, blb_ref,
             W1_ref, b1_ref, W2_ref, b2_ref,
             eg_ref, eb_ref, Wout_ref, bout_ref, out_ref):
    tok = tok_ref[0]
    fs = fsum_ref[0].astype(jnp.float32)
    etf = tok + _lnk(fs, fg_ref[0:1], fb_ref[0:1])
    x = _dot(etf.astype(jnp.bfloat16), Win_ref[...]) + bin_ref[0:1]

    for i in range(L):
        xl = _lnk(x, sg_ref[i:i + 1], sb_ref[i:i + 1]).astype(jnp.bfloat16)
        hf = (_dot(xl, Wef_ref[i]) + bef_ref[i:i + 1]).astype(jnp.bfloat16)
        hb = (_dot(xl, Web_ref[i]) + beb_ref[i:i + 1]).astype(jnp.bfloat16)
        accf = None
        accb = None
        for e in range(E):
            m = mask_ref[0, e]
            sl = slice(e * DSW, (e + 1) * DSW)
            df = _dot(m, hf[:, sl])
            db = _dot_t(m, hb[:, sl])
            accf = df if accf is None else accf + df
            accb = db if accb is None else accb + db
        xf = _lnk(accf, flg_ref[i:i + 1], flb_ref[i:i + 1])
        xb = _lnk(accb, blg_ref[i:i + 1], blb_ref[i:i + 1])
        y = _dot((xf + xb).astype(jnp.bfloat16), W1_ref[i]) + b1_ref[i:i + 1]
        y = _dot(jnp.maximum(y, 0.0).astype(jnp.bfloat16), W2_ref[i]) + b2_ref[i:i + 1]
        x = x + y

    x = _lnk(x, eg_ref[0:1], eb_ref[0:1])
    out = _dot(x.astype(jnp.bfloat16), Wout_ref[...]) + bout_ref[0:1]
    out_ref[0] = etf + out


def _tc_grid_spec():
    def bcast(rank):
        return lambda b: (0,) * rank

    full = lambda shp: pl.BlockSpec(shp, bcast(len(shp)))
    in_specs = [
        pl.BlockSpec((1, E, S, S), lambda b: (b, 0, 0, 0)),
        pl.BlockSpec((1, S, DM), lambda b: (b, 0, 0)),
        pl.BlockSpec((1, S, DM), lambda b: (b, 0, 0)),
        full((1, DM)), full((1, DM)),
        full((DM, DSW)), full((1, DSW)),
        full((L, DSW)), full((L, DSW)),
        full((L, DSW, E * DSW)), full((L, E * DSW)),
        full((L, DSW)), full((L, DSW)),
        full((L, DSW, E * DSW)), full((L, E * DSW)),
        full((L, DSW)), full((L, DSW)),
        full((L, DSW, DFF)), full((L, DFF)),
        full((L, DFF, DSW)), full((L, DSW)),
        full((1, DSW)), full((1, DSW)),
        full((DSW, DM)), full((1, DM)),
    ]
    out_spec = pl.BlockSpec((1, S, DM), lambda b: (b, 0, 0))
    return in_specs, out_spec


def _tc_dense(mask, tok3, fsum3, fg, fb, Win, bin2, sg, sb, Wef, bef, flg, flb,
              Web, beb, blg, blb, W1, b1, W2, b2, eg, eb, Wout, bout):
    in_specs, out_spec = _tc_grid_spec()
    return pl.pallas_call(
        _tc_body,
        grid=(BH,),
        in_specs=in_specs,
        out_specs=out_spec,
        out_shape=jax.ShapeDtypeStruct((BH, S, DM), jnp.float32),
        compiler_params=pltpu.CompilerParams(
            dimension_semantics=("arbitrary",),
        ),
    )(mask, tok3, fsum3, fg, fb, Win, bin2, sg, sb, Wef, bef, flg, flb,
      Web, beb, blg, blb, W1, b1, W2, b2, eg, eb, Wout, bout)


def kernel(tokens, features, edge_mask, token_table, feature_table,
           feat_ln_g, feat_ln_b, W_in, b_in, swa_ln_g, swa_ln_b,
           Wef, bef, f_ln_g, f_ln_b, Web, beb, b_ln_g, b_ln_b,
           W1, b1, W2, b2, emb_ln_g, emb_ln_b, W_out, b_out):
    tokens_i = tokens.reshape(B, S).astype(jnp.int32)
    feats_i = features.astype(jnp.int32)
    ftab_aug = jnp.concatenate(
        [feature_table, jnp.zeros((8, DM), feature_table.dtype)], axis=0)
    mask_bf = edge_mask.astype(jnp.bfloat16)

    bf = jnp.bfloat16
    weights = (feat_ln_g.reshape(1, DM), feat_ln_b.reshape(1, DM),
               W_in.astype(bf), b_in.reshape(1, DSW),
               swa_ln_g, swa_ln_b,
               Wef.astype(bf), bef, f_ln_g, f_ln_b,
               Web.astype(bf), beb, b_ln_g, b_ln_b,
               W1.astype(bf), b1, W2.astype(bf), b2,
               emb_ln_g.reshape(1, DSW), emb_ln_b.reshape(1, DSW),
               W_out.astype(bf), b_out.reshape(1, DM))

    sc_halves = []
    for h in range(HALVES):
        tokens_flat = tokens_i[h * BH:(h + 1) * BH].reshape(NTOKH)
        feat_idx2d = feats_i[h * BH:(h + 1) * BH].reshape(NW * NSUB, SUB * F)
        sc_halves.append(_sc_embed(tokens_flat, feat_idx2d, token_table,
                                   ftab_aug))

    outs = []
    for h in range(HALVES):
        tok_rows, feat_sum = sc_halves[h]
        outs.append(_tc_dense(
            mask_bf[h * BH:(h + 1) * BH],
            tok_rows.reshape(BH, S, DM), feat_sum.reshape(BH, S, DM),
            *weights))
    return jnp.concatenate(outs, axis=0)


# speedup vs baseline: 1.0231x; 1.0231x over previous
import functools

import jax
import jax.numpy as jnp
from jax import lax
from jax.experimental import pallas as pl
from jax.experimental.pallas import tpu as pltpu
from jax.experimental.pallas import tpu_sc as plsc

B, S, F = 4, 512, 8
V, DM = 32000, 768
FV = 1000
E, DSW, DFF, L = 8, 256, 1024, 2
EPS = 1e-5

NC, NS = 2, 16
NW = NC * NS
NTOK = B * S
TPW = NTOK // NW
SUB = 4
NSUB = TPW // SUB
LANES = 16
NBUF = 3


def _sc_embed_body(tok_hbm, feat_hbm, ttab_hbm, ftab_hbm,
                   tok_out, fsum_out,
                   tok_idx_v, feat_idx_v, acc_v, s0_v, s1_v, s2_v,
                   gsem0, gsem1, gsem2, tsem):
    wid = lax.axis_index("s") * NC + lax.axis_index("c")
    base = wid * TPW

    icopy = pltpu.async_copy(feat_hbm.at[pl.ds(wid * NSUB, NSUB)],
                             feat_idx_v, gsem0)
    pltpu.async_copy(tok_hbm.at[pl.ds(base, TPW)], tok_idx_v, tsem).wait()

    tok_dma = pltpu.async_copy(ttab_hbm.at[tok_idx_v], acc_v, tsem)
    icopy.wait()

    for n in range(NSUB):
        for c in range(SUB * F // LANES):
            v = feat_idx_v[n, pl.ds(c * LANES, LANES)]
            feat_idx_v[n, pl.ds(c * LANES, LANES)] = jnp.where(v == 0, FV, v)

    stages = (s0_v, s1_v, s2_v)
    gsems = (gsem0, gsem1, gsem2)

    dmas = [pltpu.async_copy(ftab_hbm.at[feat_idx_v.at[n]],
                             stages[n], gsems[n]) for n in range(NBUF - 1)]
    tok_dma.wait()
    pltpu.sync_copy(acc_v, tok_out.at[pl.ds(base, TPW)])

    for n in range(NSUB):
        if n + NBUF - 1 < NSUB:
            dmas.append(pltpu.async_copy(
                ftab_hbm.at[feat_idx_v.at[n + NBUF - 1]],
                stages[(n + NBUF - 1) % NBUF], gsems[(n + NBUF - 1) % NBUF]))
        dmas[n].wait()
        stage = stages[n % NBUF]

        def cbody(c, stage=stage, n=n):
            off = c * LANES
            for t in range(SUB):
                s = stage[t * F + 0, pl.ds(off, LANES)]
                for f in range(1, F):
                    s = s + stage[t * F + f, pl.ds(off, LANES)]
                acc_v[n * SUB + t, pl.ds(off, LANES)] = s

        plsc.parallel_loop(0, DM // LANES, 1, unroll=4)(cbody)

    pltpu.sync_copy(acc_v, fsum_out.at[pl.ds(base, TPW)])


def _sc_embed(tokens_flat, feat_idx2d, token_table, ftab_aug):
    mesh = plsc.VectorSubcoreMesh(core_axis_name="c", subcore_axis_name="s",
                                  num_cores=NC, num_subcores=NS)
    k = pl.kernel(
        _sc_embed_body,
        out_type=[jax.ShapeDtypeStruct((NTOK, DM), jnp.float32),
                  jax.ShapeDtypeStruct((NTOK, DM), jnp.float32)],
        mesh=mesh,
        scratch_types=[
            pltpu.VMEM((TPW,), jnp.int32),
            pltpu.VMEM((NSUB, SUB * F), jnp.int32),
            pltpu.VMEM((TPW, DM), jnp.float32),
            pltpu.VMEM((SUB * F, DM), jnp.float32),
            pltpu.VMEM((SUB * F, DM), jnp.float32),
            pltpu.VMEM((SUB * F, DM), jnp.float32),
            pltpu.SemaphoreType.DMA,
            pltpu.SemaphoreType.DMA,
            pltpu.SemaphoreType.DMA,
            pltpu.SemaphoreType.DMA,
        ],
    )
    return k(tokens_flat, feat_idx2d, token_table, ftab_aug)


def _lnk(x, g, b):
    mu = jnp.mean(x, axis=-1, keepdims=True)
    msq = jnp.mean(x * x, axis=-1, keepdims=True)
    var = msq - mu * mu
    return (x - mu) * lax.rsqrt(var + EPS) * g + b


def _dot(a, b):
    return lax.dot_general(a, b, (((1,), (0,)), ((), ())),
                           preferred_element_type=jnp.float32)


def _dot_t(a, b):
    return lax.dot_general(a, b, (((0,), (0,)), ((), ())),
                           preferred_element_type=jnp.float32)


def _tc_body(mask_ref, tok_ref, fsum_ref, fg_ref, fb_ref,
             Win_ref, bin_ref, sg_ref, sb_ref,
             Wef_ref, bef_ref, flg_ref, flb_ref,
             Web_ref, beb_ref, blg_ref, blb_ref,
             W1_ref, b1_ref, W2_ref, b2_ref,
             eg_ref, eb_ref, Wout_ref, bout_ref, out_ref):
    tok = tok_ref[0]
    fs = fsum_ref[0].astype(jnp.float32)
    etf = tok + _lnk(fs, fg_ref[0:1], fb_ref[0:1])
    x = _dot(etf.astype(jnp.bfloat16), Win_ref[...]) + bin_ref[0:1]

    for i in range(L):
        xl = _lnk(x, sg_ref[i:i + 1], sb_ref[i:i + 1]).astype(jnp.bfloat16)
        hf = (_dot(xl, Wef_ref[i]) + bef_ref[i:i + 1]).astype(jnp.bfloat16)
        hb = (_dot(xl, Web_ref[i]) + beb_ref[i:i + 1]).astype(jnp.bfloat16)
        accf = None
        accb = None
        for e in range(E):
            m = mask_ref[0, e]
            sl = slice(e * DSW, (e + 1) * DSW)
            df = _dot(m, hf[:, sl])
            db = _dot_t(m, hb[:, sl])
            accf = df if accf is None else accf + df
            accb = db if accb is None else accb + db
        xf = _lnk(accf, flg_ref[i:i + 1], flb_ref[i:i + 1])
        xb = _lnk(accb, blg_ref[i:i + 1], blb_ref[i:i + 1])
        y = _dot((xf + xb).astype(jnp.bfloat16), W1_ref[i]) + b1_ref[i:i + 1]
        y = _dot(jnp.maximum(y, 0.0).astype(jnp.bfloat16), W2_ref[i]) + b2_ref[i:i + 1]
        x = x + y

    x = _lnk(x, eg_ref[0:1], eb_ref[0:1])
    out = _dot(x.astype(jnp.bfloat16), Wout_ref[...]) + bout_ref[0:1]
    out_ref[0] = etf + out


def _tc_grid_spec():
    def bcast(rank):
        return lambda b: (0,) * rank

    full = lambda shp: pl.BlockSpec(shp, bcast(len(shp)))
    in_specs = [
        pl.BlockSpec((1, E, S, S), lambda b: (b, 0, 0, 0)),
        pl.BlockSpec((1, S, DM), lambda b: (b, 0, 0)),
        pl.BlockSpec((1, S, DM), lambda b: (b, 0, 0)),
        full((1, DM)), full((1, DM)),
        full((DM, DSW)), full((1, DSW)),
        full((L, DSW)), full((L, DSW)),
        full((L, DSW, E * DSW)), full((L, E * DSW)),
        full((L, DSW)), full((L, DSW)),
        full((L, DSW, E * DSW)), full((L, E * DSW)),
        full((L, DSW)), full((L, DSW)),
        full((L, DSW, DFF)), full((L, DFF)),
        full((L, DFF, DSW)), full((L, DSW)),
        full((1, DSW)), full((1, DSW)),
        full((DSW, DM)), full((1, DM)),
    ]
    out_spec = pl.BlockSpec((1, S, DM), lambda b: (b, 0, 0))
    return in_specs, out_spec


def _tc_dense(mask, tok3, fsum3, fg, fb, Win, bin2, sg, sb, Wef, bef, flg, flb,
              Web, beb, blg, blb, W1, b1, W2, b2, eg, eb, Wout, bout):
    in_specs, out_spec = _tc_grid_spec()
    return pl.pallas_call(
        _tc_body,
        grid=(B,),
        in_specs=in_specs,
        out_specs=out_spec,
        out_shape=jax.ShapeDtypeStruct((B, S, DM), jnp.float32),
        compiler_params=pltpu.CompilerParams(
            dimension_semantics=("arbitrary",),
        ),
    )(mask, tok3, fsum3, fg, fb, Win, bin2, sg, sb, Wef, bef, flg, flb,
      Web, beb, blg, blb, W1, b1, W2, b2, eg, eb, Wout, bout)


def kernel(tokens, features, edge_mask, token_table, feature_table,
           feat_ln_g, feat_ln_b, W_in, b_in, swa_ln_g, swa_ln_b,
           Wef, bef, f_ln_g, f_ln_b, Web, beb, b_ln_g, b_ln_b,
           W1, b1, W2, b2, emb_ln_g, emb_ln_b, W_out, b_out):
    tokens_flat = tokens.reshape(NTOK).astype(jnp.int32)
    feat_idx2d = features.reshape(NW * NSUB, SUB * F).astype(jnp.int32)
    ftab_aug = jnp.concatenate(
        [feature_table, jnp.zeros((8, DM), feature_table.dtype)], axis=0)

    tok_rows, feat_sum = _sc_embed(tokens_flat, feat_idx2d, token_table,
                                   ftab_aug)

    bf = jnp.bfloat16
    out = _tc_dense(
        edge_mask.astype(bf),
        tok_rows.reshape(B, S, DM), feat_sum.reshape(B, S, DM),
        feat_ln_g.reshape(1, DM), feat_ln_b.reshape(1, DM),
        W_in.astype(bf), b_in.reshape(1, DSW),
        swa_ln_g, swa_ln_b,
        Wef.astype(bf), bef, f_ln_g, f_ln_b,
        Web.astype(bf), beb, b_ln_g, b_ln_b,
        W1.astype(bf), b1, W2.astype(bf), b2,
        emb_ln_g.reshape(1, DSW), emb_ln_b.reshape(1, DSW),
        W_out.astype(bf), b_out.reshape(1, DM))
    return out
